# Initial kernel scaffold; baseline (speedup 1.0000x reference)
import jax
import jax.numpy as jnp
from jax.experimental import pallas as pl


def kernel(class_logits, box_regression, proposals, image_h, image_w):
    raise NotImplementedError("write your pallas kernel here")


# speedup vs baseline: 8.1837x; 8.1837x over previous
import math

import jax
import jax.numpy as jnp
from jax.experimental import pallas as pl

_SCORE_T = 0.05
_NMS_T = 0.5
_DETS = 100
_MINSZ = 0.01
_CLIP = float(math.log(1000.0 / 16.0))
_TOPK = 1000
_K = 1024


def _stage1(num_classes, hw_ref, lg_ref, dx_ref, dy_ref, dw_ref, dh_ref,
            pr_ref, ms_ref, x1_ref, y1_ref, x2_ref, y2_ref, mx_ref):
    wf = hw_ref[0, 0]
    hf = hw_ref[0, 1]
    p = pr_ref[:]
    pw = p[:, 2:3] - p[:, 0:1]
    ph = p[:, 3:4] - p[:, 1:2]
    cx = p[:, 0:1] + 0.5 * pw
    cy = p[:, 1:2] + 0.5 * ph
    lg = lg_ref[:]
    m = jnp.max(lg, axis=1, keepdims=True)
    e = jnp.exp(lg - m)
    s = e / jnp.sum(e, axis=1, keepdims=True)
    dx = dx_ref[:] / 10.0
    dy = dy_ref[:] / 10.0
    dw = jnp.minimum(dw_ref[:] / 5.0, _CLIP)
    dh = jnp.minimum(dh_ref[:] / 5.0, _CLIP)
    px = dx * pw + cx
    py = dy * ph + cy
    bw = jnp.exp(dw) * pw
    bh = jnp.exp(dh) * ph
    x1 = jnp.clip(px - 0.5 * bw, 0.0, wf)
    y1 = jnp.clip(py - 0.5 * bh, 0.0, hf)
    x2 = jnp.clip(px + 0.5 * bw, 0.0, wf)
    y2 = jnp.clip(py + 0.5 * bh, 0.0, hf)
    colid = jax.lax.broadcasted_iota(jnp.int32, lg.shape, 1)
    clsm = (colid >= 1) & (colid < num_classes)
    valid = (s > _SCORE_T) & (x2 - x1 >= _MINSZ) & (y2 - y1 >= _MINSZ) & clsm
    ms_ref[:] = jnp.where(valid, s, -1.0)
    x1_ref[:] = x1
    y1_ref[:] = y1
    x2_ref[:] = x2
    y2_ref[:] = y2
    big = jnp.maximum(jnp.maximum(x1, x2), jnp.maximum(y1, y2))
    cur = jnp.max(jnp.where(clsm, big, -jnp.inf), axis=0, keepdims=True)

    @pl.when(pl.program_id(0) == 0)
    def _init():
        mx_ref[:] = cur

    @pl.when(pl.program_id(0) != 0)
    def _acc():
        mx_ref[:] = jnp.maximum(mx_ref[:], cur)


def _stage2(mx_ref, scc_ref, scr_ref, x1c_ref, y1c_ref, x2c_ref, y2c_ref,
            x1r_ref, y1r_ref, x2r_ref, y2r_ref, lc_ref, lr_ref, kept_ref):
    off = mx_ref[0, 0] + 1.0
    oc = lc_ref[:] * off
    orow = lr_ref[:] * off
    rx1 = x1c_ref[:] + oc
    ry1 = y1c_ref[:] + oc
    rx2 = x2c_ref[:] + oc
    ry2 = y2c_ref[:] + oc
    cx1 = x1r_ref[:] + orow
    cy1 = y1r_ref[:] + orow
    cx2 = x2r_ref[:] + orow
    cy2 = y2r_ref[:] + orow
    area_c = (rx2 - rx1) * (ry2 - ry1)
    area_r = (cx2 - cx1) * (cy2 - cy1)
    iw = jnp.maximum(jnp.minimum(rx2, cx2) - jnp.maximum(rx1, cx1), 0.0)
    ih = jnp.maximum(jnp.minimum(ry2, cy2) - jnp.maximum(ry1, cy1), 0.0)
    inter = iw * ih
    iou = inter / (area_c + area_r - inter + 1e-9)
    hit = iou > _NMS_T
    ri = jax.lax.broadcasted_iota(jnp.int32, iou.shape, 0)
    ci = jax.lax.broadcasted_iota(jnp.int32, iou.shape, 1)
    sup_fwd = jnp.where(hit & (ri < ci), 1.0, 0.0)
    sup_bwd = jnp.where(hit & (ci < ri), 1.0, 0.0)
    validc = scc_ref[:] > 0.0
    validr = scr_ref[:] > 0.0
    k0 = jnp.where(validc, 1.0, 0.0)

    def body(carry):
        k, _, t = carry
        srow = jnp.sum(sup_fwd * k, axis=0, keepdims=True)
        krow = jnp.where(validr & (srow < 0.5), 1.0, 0.0)
        scol = jnp.sum(sup_bwd * krow, axis=1, keepdims=True)
        knew = jnp.where(validc & (scol < 0.5), 1.0, 0.0)
        return knew, jnp.any(knew != k), t + 1

    def cond(carry):
        _, changed, t = carry
        return changed & (t < _K)

    kfin, _, _ = jax.lax.while_loop(
        cond, body, (k0, jnp.bool_(True), jnp.int32(0)))
    kept_ref[:] = jnp.where(kfin > 0.5, scc_ref[:], -1.0)


def kernel(class_logits, box_regression, proposals, image_h, image_w):
    N, C = class_logits.shape
    CP = 128
    R = 1000
    G = N // R
    f32 = jnp.float32

    wf = jnp.asarray(image_w, f32)
    hf = jnp.asarray(image_h, f32)
    hw = jnp.stack([wf, hf]).reshape(1, 2)

    lg = jnp.pad(class_logits, ((0, 0), (0, CP - C)), constant_values=-1e30)
    rel = box_regression.reshape(N, C, 4)
    padc = lambda a: jnp.pad(a, ((0, 0), (0, CP - C)))
    dx = padc(rel[:, :, 0])
    dy = padc(rel[:, :, 1])
    dw = padc(rel[:, :, 2])
    dh = padc(rel[:, :, 3])

    row_spec = pl.BlockSpec((R, CP), lambda i: (i, 0))
    ms, x1, y1, x2, y2, mx = pl.pallas_call(
        lambda *refs: _stage1(C, *refs),
        grid=(G,),
        in_specs=[
            pl.BlockSpec((1, 2), lambda i: (0, 0)),
            row_spec, row_spec, row_spec, row_spec, row_spec,
            pl.BlockSpec((R, 4), lambda i: (i, 0)),
        ],
        out_specs=[
            row_spec, row_spec, row_spec, row_spec, row_spec,
            pl.BlockSpec((1, CP), lambda i: (0, 0)),
        ],
        out_shape=[
            jax.ShapeDtypeStruct((N, CP), f32),
            jax.ShapeDtypeStruct((N, CP), f32),
            jax.ShapeDtypeStruct((N, CP), f32),
            jax.ShapeDtypeStruct((N, CP), f32),
            jax.ShapeDtypeStruct((N, CP), f32),
            jax.ShapeDtypeStruct((1, CP), f32),
        ],
    )(hw, lg, dx, dy, dw, dh, proposals)
    mx = jnp.max(mx).reshape(1, 1)

    ms_flat = ms[:, 1:C].reshape(-1)
    top_s, idx = jax.lax.top_k(ms_flat, _TOPK)
    ci_sel = idx % (C - 1) + 1
    labels = ci_sel
    flat2 = (idx // (C - 1)) * CP + ci_sel
    bx1 = x1.reshape(-1)[flat2]
    by1 = y1.reshape(-1)[flat2]
    bx2 = x2.reshape(-1)[flat2]
    by2 = y2.reshape(-1)[flat2]

    padk = lambda v, val: jnp.pad(v, (0, _K - _TOPK), constant_values=val)
    sc_p = padk(top_s, -1.0)
    x1_p = padk(bx1, 0.0)
    y1_p = padk(by1, 0.0)
    x2_p = padk(bx2, 0.0)
    y2_p = padk(by2, 0.0)
    lab_p = padk(labels.astype(f32), 0.0)
    col = lambda v: v.reshape(_K, 1)
    row = lambda v: v.reshape(1, _K)

    col_spec = pl.BlockSpec((_K, 1), lambda: (0, 0))
    row_sp = pl.BlockSpec((1, _K), lambda: (0, 0))
    kept = pl.pallas_call(
        _stage2,
        in_specs=[
            pl.BlockSpec((1, 1), lambda: (0, 0)),
            col_spec, row_sp,
            col_spec, col_spec, col_spec, col_spec,
            row_sp, row_sp, row_sp, row_sp,
            col_spec, row_sp,
        ],
        out_specs=col_spec,
        out_shape=jax.ShapeDtypeStruct((_K, 1), f32),
    )(mx, col(sc_p), row(sc_p),
      col(x1_p), col(y1_p), col(x2_p), col(y2_p),
      row(x1_p), row(y1_p), row(x2_p), row(y2_p),
      col(lab_p), row(lab_p))

    kept_s = kept.reshape(-1)[:_TOPK]
    final_scores, fidx = jax.lax.top_k(kept_s, _DETS)
    fb = jnp.stack([bx1[fidx], by1[fidx], bx2[fidx], by2[fidx]], axis=-1)
    fl = labels[fidx]
    fvalid = final_scores > 0.0
    areas = (fb[:, 2] - fb[:, 0]) * (fb[:, 3] - fb[:, 1])
    fvalid = fvalid & (areas >= 0.0)
    final_boxes = jnp.where(fvalid[:, None], fb, 0.0)
    final_scores = jnp.where(fvalid, final_scores, 0.0)
    final_labels = jnp.where(fvalid, fl, 0)
    return final_boxes, final_scores, final_labels


# speedup vs baseline: 23.2083x; 2.8359x over previous
import functools
import math

import jax
import jax.numpy as jnp
from jax import lax
from jax.experimental import pallas as pl
from jax.experimental.pallas import tpu as pltpu
from jax.experimental.pallas import tpu_sc as plsc

_SCORE_T = 0.05
_NMS_T = 0.5
_DETS = 100
_MINSZ = 0.01
_CLIP = float(math.log(1000.0 / 16.0))
_TOPK = 1000
_K = 1024
_W = 32
_ONE_BITS = 1065353216


def _stage1(num_classes, hw_ref, lg_ref, dx_ref, dy_ref, dw_ref, dh_ref,
            pr_ref, ms_ref, x1_ref, y1_ref, x2_ref, y2_ref, mx_ref):
    wf = hw_ref[0, 0]
    hf = hw_ref[0, 1]
    p = pr_ref[:]
    pw = p[:, 2:3] - p[:, 0:1]
    ph = p[:, 3:4] - p[:, 1:2]
    cx = p[:, 0:1] + 0.5 * pw
    cy = p[:, 1:2] + 0.5 * ph
    lg = lg_ref[:]
    m = jnp.max(lg, axis=1, keepdims=True)
    e = jnp.exp(lg - m)
    s = e / jnp.sum(e, axis=1, keepdims=True)
    dx = dx_ref[:] / 10.0
    dy = dy_ref[:] / 10.0
    dw = jnp.minimum(dw_ref[:] / 5.0, _CLIP)
    dh = jnp.minimum(dh_ref[:] / 5.0, _CLIP)
    px = dx * pw + cx
    py = dy * ph + cy
    bw = jnp.exp(dw) * pw
    bh = jnp.exp(dh) * ph
    x1 = jnp.clip(px - 0.5 * bw, 0.0, wf)
    y1 = jnp.clip(py - 0.5 * bh, 0.0, hf)
    x2 = jnp.clip(px + 0.5 * bw, 0.0, wf)
    y2 = jnp.clip(py + 0.5 * bh, 0.0, hf)
    colid = jax.lax.broadcasted_iota(jnp.int32, lg.shape, 1)
    clsm = (colid >= 1) & (colid < num_classes)
    valid = (s > _SCORE_T) & (x2 - x1 >= _MINSZ) & (y2 - y1 >= _MINSZ) & clsm
    ms_ref[:] = jnp.where(valid, s, -1.0)
    x1_ref[:] = x1
    y1_ref[:] = y1
    x2_ref[:] = x2
    y2_ref[:] = y2
    big = jnp.maximum(jnp.maximum(x1, x2), jnp.maximum(y1, y2))
    cur = jnp.max(jnp.where(clsm, big, -jnp.inf), axis=0, keepdims=True)

    @pl.when(pl.program_id(0) == 0)
    def _init():
        mx_ref[:] = cur

    @pl.when(pl.program_id(0) != 0)
    def _acc():
        mx_ref[:] = jnp.maximum(mx_ref[:], cur)


def _threshold(ms_ref, t_ref, xt_ref):
    msv = ms_ref[:]

    def count_gt(bits):
        tv = jax.lax.bitcast_convert_type(
            jnp.full((1, 128), bits, jnp.int32), jnp.float32)
        return jnp.sum((msv > tv).astype(jnp.int32))

    c0 = count_gt(jnp.int32(0))

    def bits_body(_, carry):
        lo, hi = carry
        mid = (lo + hi) // 2
        c = count_gt(mid)
        take_hi = c >= _TOPK
        return jnp.where(take_hi, mid, lo), jnp.where(take_hi, hi, mid)

    _, hi_bits = lax.fori_loop(
        0, 31, bits_body, (jnp.int32(0), jnp.int32(_ONE_BITS)))
    t_bits = jnp.where(c0 <= _TOPK, jnp.int32(0), hi_bits)
    tv = jax.lax.bitcast_convert_type(
        jnp.full((1, 128), t_bits, jnp.int32), jnp.float32)
    m_cnt = jnp.sum((msv > tv).astype(jnp.int32))
    e_need = _TOPK - m_cnt
    eq = msv == tv
    fl = (jax.lax.broadcasted_iota(jnp.int32, msv.shape, 0) * 128
          + jax.lax.broadcasted_iota(jnp.int32, msv.shape, 1))

    def idx_body(_, carry):
        lo, hi = carry
        mid = (lo + hi) // 2
        c = jnp.sum((eq & (fl <= mid)).astype(jnp.int32))
        take_hi = c >= e_need
        return jnp.where(take_hi, lo, mid), jnp.where(take_hi, mid, hi)

    ntot = msv.shape[0] * msv.shape[1]
    _, xt_hi = lax.fori_loop(
        0, 21, idx_body, (jnp.int32(-1), jnp.int32(ntot - 1)))
    xt = jnp.where(c0 <= _TOPK, jnp.int32(-1), xt_hi)
    t_ref[:] = tv
    xt_ref[:] = jnp.full((1, 128), xt)


def _sc_body(ms_hbm, t_hbm, xt_hbm, lists_hbm, cnts_hbm,
             in_v, out_v, t_v, xt_v, cnt_v):
    nper = 20000
    wid = lax.axis_index("s") * 2 + lax.axis_index("c")
    base = wid * nper
    pltpu.sync_copy(ms_hbm.at[pl.ds(base, nper)], in_v)
    pltpu.sync_copy(t_hbm, t_v)
    pltpu.sync_copy(xt_hbm, xt_v)
    tvec = t_v[...]
    xtvec = xt_v[...]
    lane = jax.lax.iota(jnp.int32, 16)
    lane_base = lane * _K

    def it(i, pos):
        v = in_v[pl.ds(i * 16, 16)]
        gi = lane + (base + i * 16)
        m = (v > tvec) | ((v == tvec) & (gi <= xtvec))
        dst = jnp.where(m, lane_base + pos, 16 * _K + lane)
        plsc.store_scatter(out_v, [dst], gi)
        return pos + jnp.where(m, 1, 0)

    pos = lax.fori_loop(0, nper // 16, it, jnp.zeros((16,), jnp.int32))
    cnt_v[...] = pos
    pltpu.sync_copy(cnt_v, cnts_hbm.at[pl.ds(wid * 16, 16)])
    pltpu.sync_copy(out_v.at[pl.ds(0, 16 * _K)],
                    lists_hbm.at[pl.ds(wid * 16 * _K, 16 * _K)])


def _stage2(mx_ref, scc_ref, scr_ref, x1c_ref, y1c_ref, x2c_ref, y2c_ref,
            x1r_ref, y1r_ref, x2r_ref, y2r_ref, lc_ref, lr_ref,
            fic_ref, fir_ref, kept_ref):
    off = mx_ref[0, 0] + 1.0
    oc = lc_ref[:] * off
    orow = lr_ref[:] * off
    rx1 = x1c_ref[:] + oc
    ry1 = y1c_ref[:] + oc
    rx2 = x2c_ref[:] + oc
    ry2 = y2c_ref[:] + oc
    cx1 = x1r_ref[:] + orow
    cy1 = y1r_ref[:] + orow
    cx2 = x2r_ref[:] + orow
    cy2 = y2r_ref[:] + orow
    area_c = (rx2 - rx1) * (ry2 - ry1)
    area_r = (cx2 - cx1) * (cy2 - cy1)
    iw = jnp.maximum(jnp.minimum(rx2, cx2) - jnp.maximum(rx1, cx1), 0.0)
    ih = jnp.maximum(jnp.minimum(ry2, cy2) - jnp.maximum(ry1, cy1), 0.0)
    inter = iw * ih
    iou = inter / (area_c + area_r - inter + 1e-9)
    hit = iou > _NMS_T
    scc = scc_ref[:]
    scr = scr_ref[:]
    fic = fic_ref[:]
    fir = fir_ref[:]
    prec_fwd = (scc > scr) | ((scc == scr) & (fic < fir))
    prec_bwd = (scr > scc) | ((scr == scc) & (fir < fic))
    sup_fwd = jnp.where(hit & prec_fwd, 1.0, 0.0)
    sup_bwd = jnp.where(hit & prec_bwd, 1.0, 0.0)
    validc = scc > 0.0
    validr = scr > 0.0
    k0 = jnp.where(validc, 1.0, 0.0)

    def body(carry):
        k, _, t = carry
        srow = jnp.sum(sup_fwd * k, axis=0, keepdims=True)
        krow = jnp.where(validr & (srow < 0.5), 1.0, 0.0)
        scol = jnp.sum(sup_bwd * krow, axis=1, keepdims=True)
        knew = jnp.where(validc & (scol < 0.5), 1.0, 0.0)
        return knew, jnp.any(knew != k), t + 1

    def cond(carry):
        _, changed, t = carry
        return changed & (t < _K)

    kfin, _, _ = jax.lax.while_loop(
        cond, body, (k0, jnp.bool_(True), jnp.int32(0)))
    kept_ref[:] = jnp.where(kfin > 0.5, scc, -1.0)


def kernel(class_logits, box_regression, proposals, image_h, image_w):
    N, C = class_logits.shape
    CP = 128
    R = 1000
    G = N // R
    f32 = jnp.float32
    i32 = jnp.int32

    wf = jnp.asarray(image_w, f32)
    hf = jnp.asarray(image_h, f32)
    hw = jnp.stack([wf, hf]).reshape(1, 2)

    lg = jnp.pad(class_logits, ((0, 0), (0, CP - C)), constant_values=-1e30)
    rel = box_regression.reshape(N, C, 4)
    padc = lambda a: jnp.pad(a, ((0, 0), (0, CP - C)))
    dx = padc(rel[:, :, 0])
    dy = padc(rel[:, :, 1])
    dw = padc(rel[:, :, 2])
    dh = padc(rel[:, :, 3])

    row_spec = pl.BlockSpec((R, CP), lambda i: (i, 0))
    ms, x1, y1, x2, y2, mx = pl.pallas_call(
        lambda *refs: _stage1(C, *refs),
        grid=(G,),
        in_specs=[
            pl.BlockSpec((1, 2), lambda i: (0, 0)),
            row_spec, row_spec, row_spec, row_spec, row_spec,
            pl.BlockSpec((R, 4), lambda i: (i, 0)),
        ],
        out_specs=[
            row_spec, row_spec, row_spec, row_spec, row_spec,
            pl.BlockSpec((1, CP), lambda i: (0, 0)),
        ],
        out_shape=[
            jax.ShapeDtypeStruct((N, CP), f32),
            jax.ShapeDtypeStruct((N, CP), f32),
            jax.ShapeDtypeStruct((N, CP), f32),
            jax.ShapeDtypeStruct((N, CP), f32),
            jax.ShapeDtypeStruct((N, CP), f32),
            jax.ShapeDtypeStruct((1, CP), f32),
        ],
    )(hw, lg, dx, dy, dw, dh, proposals)
    mx = jnp.max(mx).reshape(1, 1)

    t_arr, xt_arr = pl.pallas_call(
        _threshold,
        out_shape=[
            jax.ShapeDtypeStruct((1, CP), f32),
            jax.ShapeDtypeStruct((1, CP), i32),
        ],
    )(ms)
    t16 = jnp.broadcast_to(t_arr[0, :16], (16,))
    xt16 = jnp.broadcast_to(xt_arr[0, :16], (16,))

    mesh = plsc.VectorSubcoreMesh(core_axis_name="c", subcore_axis_name="s")
    sc_compact = functools.partial(
        pl.kernel,
        mesh=mesh,
        compiler_params=pltpu.CompilerParams(needs_layout_passes=False),
        out_type=[
            jax.ShapeDtypeStruct((_W * 16 * _K,), i32),
            jax.ShapeDtypeStruct((_W * 16,), i32),
        ],
        scratch_types=[
            pltpu.VMEM((20000,), f32),
            pltpu.VMEM((16 * _K + 16,), i32),
            pltpu.VMEM((16,), f32),
            pltpu.VMEM((16,), i32),
            pltpu.VMEM((16,), i32),
        ],
    )(_sc_body)
    lists_o, cnts_o = sc_compact(ms.reshape(-1), t16, xt16)

    nlists = _W * 16
    cnts = cnts_o
    ends = jnp.cumsum(cnts)
    total = ends[nlists - 1]
    r = jnp.arange(_K, dtype=i32)
    w = jnp.sum((ends[None, :] <= r[:, None]).astype(i32), axis=1)
    wc = jnp.minimum(w, nlists - 1)
    starts = ends[wc] - cnts[wc]
    src = wc * _K + (r - starts)
    valid_r = r < total
    flatidx = jnp.where(valid_r, lists_o[jnp.where(valid_r, src, 0)], 0)

    sc1024 = ms.reshape(-1)[flatidx]
    labels = flatidx % CP
    bx1 = x1.reshape(-1)[flatidx]
    by1 = y1.reshape(-1)[flatidx]
    bx2 = x2.reshape(-1)[flatidx]
    by2 = y2.reshape(-1)[flatidx]

    col = lambda v: v.reshape(_K, 1)
    row = lambda v: v.reshape(1, _K)
    col_spec = pl.BlockSpec((_K, 1), lambda: (0, 0))
    row_sp = pl.BlockSpec((1, _K), lambda: (0, 0))
    kept = pl.pallas_call(
        _stage2,
        in_specs=[
            pl.BlockSpec((1, 1), lambda: (0, 0)),
            col_spec, row_sp,
            col_spec, col_spec, col_spec, col_spec,
            row_sp, row_sp, row_sp, row_sp,
            col_spec, row_sp,
            col_spec, row_sp,
        ],
        out_specs=col_spec,
        out_shape=jax.ShapeDtypeStruct((_K, 1), f32),
    )(mx, col(sc1024), row(sc1024),
      col(bx1), col(by1), col(bx2), col(by2),
      row(bx1), row(by1), row(bx2), row(by2),
      col(labels.astype(f32)), row(labels.astype(f32)),
      col(flatidx), row(flatidx))

    kept_s = kept.reshape(-1)
    final_scores, fidx = jax.lax.top_k(kept_s, _DETS)
    fb = jnp.stack([bx1[fidx], by1[fidx], bx2[fidx], by2[fidx]], axis=-1)
    fl = labels[fidx]
    fvalid = final_scores > 0.0
    areas = (fb[:, 2] - fb[:, 0]) * (fb[:, 3] - fb[:, 1])
    fvalid = fvalid & (areas >= 0.0)
    final_boxes = jnp.where(fvalid[:, None], fb, 0.0)
    final_scores = jnp.where(fvalid, final_scores, 0.0)
    final_labels = jnp.where(fvalid, fl, 0)
    return final_boxes, final_scores, final_labels
